# baseline (speedup 1.0000x reference)
import functools

import jax
import jax.numpy as jnp
from jax import lax
from jax.experimental import pallas as pl
from jax.experimental.pallas import tpu as pltpu
from jax.experimental.pallas import tpu_sc as plsc

_B = 16384
_D = 16
_NC = 2
_NS = 16
_NW = _NC * _NS
_BPW = _B // _NW
_CHUNK = 128
_NIDX = 3 * _BPW
_NCHUNK = _NIDX // _CHUNK

_LN2 = 0.6931471805599453
_SQRT2 = 1.4142135623730951


def _log16(x):
    xi = lax.bitcast_convert_type(x, jnp.int32)
    e = jnp.right_shift(xi, 23) - 127
    m = lax.bitcast_convert_type(
        jnp.bitwise_or(jnp.bitwise_and(xi, 0x007FFFFF), 0x3F800000), jnp.float32)
    big = m > _SQRT2
    m = jnp.where(big, m * 0.5, m)
    ef = e.astype(jnp.float32) + jnp.where(big, 1.0, 0.0)
    z = (m - 1.0) / (m + 1.0)
    z2 = z * z
    p = z * (2.0 + z2 * (0.66666667 + z2 * (0.4 + z2 * 0.28571429)))
    return ef * _LN2 + p


_mesh = plsc.VectorSubcoreMesh(core_axis_name="c", subcore_axis_name="s")


@functools.partial(
    pl.kernel,
    mesh=_mesh,
    compiler_params=pltpu.CompilerParams(
        needs_layout_passes=False, use_tc_tiling_on_sc=False),
    out_type=jax.ShapeDtypeStruct((_B,), jnp.float32),
    scratch_types=[
        pltpu.VMEM((_NCHUNK, _CHUNK), jnp.int32),
        pltpu.VMEM((_NIDX, _D), jnp.float32),
        pltpu.VMEM((_BPW,), jnp.float32),
        pltpu.SemaphoreType.DMA,
    ],
)
def _tste_sc(idx_hbm, table_hbm, out_hbm, idx_v, rows_v, out_v, sem):
    wid = lax.axis_index("s") * _NC + lax.axis_index("c")

    pltpu.sync_copy(idx_hbm.at[wid], idx_v)
    copies = [
        pltpu.async_copy(
            table_hbm.at[idx_v.at[j]],
            rows_v.at[pl.ds(j * _CHUNK, _CHUNK)],
            sem,
        )
        for j in range(_NCHUNK)
    ]
    for cp in copies:
        cp.wait()

    lane = lax.iota(jnp.int32, 16)

    def group(g, carry):
        rh = lane + g * 16
        rw = rh + _BPW
        rl = rh + 2 * _BPW
        accw = jnp.zeros((16,), jnp.float32)
        accl = jnp.zeros((16,), jnp.float32)
        for d in range(_D):
            col = jnp.full((16,), d, jnp.int32)
            hd = plsc.load_gather(rows_v, [rh, col])
            wd = plsc.load_gather(rows_v, [rw, col])
            ld = plsc.load_gather(rows_v, [rl, col])
            dw = hd - wd
            dl = hd - ld
            accw = accw + dw * dw
            accl = accl + dl * dl
        x = 1.0 + (1.0 + accw) / (1.0 + accl)
        out_v[pl.ds(g * 16, 16)] = _log16(x)
        return carry

    lax.fori_loop(0, _BPW // 16, group, 0)

    base = pl.multiple_of(wid * _BPW, 8)
    pltpu.sync_copy(out_v, out_hbm.at[pl.ds(base, _BPW)])


def kernel(h_w_l, embedding):
    idx = (
        h_w_l.astype(jnp.int32)
        .reshape(_NW, _BPW, 3)
        .transpose(0, 2, 1)
        .reshape(_NW, _NCHUNK, _CHUNK)
    )
    return _tste_sc(idx, embedding)
